# baseline (speedup 1.0000x reference)
import functools

import jax
import jax.numpy as jnp
from jax import lax
from jax.experimental import pallas as pl
from jax.experimental.pallas import tpu as pltpu
from jax.experimental.pallas import tpu_sc as plsc

N_LEMMAS = 100000
N_CONCEPTS = 64
D_PHON = 64
B = 16384
LR = 0.05
THETA_A = 0.3

NC = 2
NS = 16
NW = NC * NS
PASSES = 10
RS = 312
ROWS_PER_W = PASSES * RS
TAIL = N_LEMMAS - NW * ROWS_PER_W
MC = 64
EID_BITS = 14
IDX_BLK = 2048


def _body(idx_hbm, wcl_hbm, wlp_hbm, cp_hbm, out_hbm,
          idx_stage, span_buf, pass_buf, eid_buf, rloc_buf, gate_buf,
          tree_buf, cp_chunk, wcl_chunk, wlp_chunk,
          sem_a, sem_b, sem_c):
    wid = lax.axis_index("s") * NC + lax.axis_index("c")
    span_lo = wid * ROWS_PER_W
    is_last = wid == NW - 1
    span_hi = span_lo + ROWS_PER_W + (wid // (NW - 1)) * TAIL
    iota = lax.iota(jnp.int32, 16)

    tree_buf[pl.ds(0, 16)] = jnp.zeros((16,), jnp.int32)

    def prefix16(ind):
        tree_buf[pl.ds(16, 16)] = ind
        v = ind
        for sh in (1, 2, 4):
            v = v + tree_buf[pl.ds(16 - sh, 16)]
            tree_buf[pl.ds(16, 16)] = v
        v = v + tree_buf[pl.ds(16 - 8, 16)]
        return v

    def in_range01(x, lo, hi):
        d1 = lax.shift_right_arithmetic(x - lo, 31)
        d2 = lax.shift_right_arithmetic(hi - 1 - x, 31)
        return (1 + d1) & (1 + d2)

    def compress(ind, payload, dst, ns):
        pref = prefix16(ind)
        cnt = pref[15]

        @pl.when(cnt > 0)
        def _():
            pos = (ns + pref - 1) * ind + (1 - ind) * B
            plsc.store_scatter(dst, [pos], payload)

        return ns + cnt

    def blk_body(b, n_span):
        pltpu.sync_copy(idx_hbm.at[pl.ds(b * IDX_BLK, IDX_BLK)], idx_stage)

        def vec_body(v, ns):
            ivec = idx_stage[pl.ds(v * 16, 16)]
            ind = in_range01(ivec, span_lo, span_hi)
            packed = (ivec << EID_BITS) | (b * IDX_BLK + v * 16 + iota)
            return compress(ind, packed, span_buf, ns)

        return lax.fori_loop(0, IDX_BLK // 16, vec_body, n_span)

    n_span = lax.fori_loop(0, B // IDX_BLK, blk_body, 0)

    def do_pass(lo, rs):
        wcl_dst = wcl_chunk if rs == RS else wcl_chunk.at[pl.ds(0, rs)]
        wlp_dst = wlp_chunk if rs == RS else wlp_chunk.at[pl.ds(0, rs)]
        cp_wcl = pltpu.async_copy(wcl_hbm.at[pl.ds(lo, rs)], wcl_dst, sem_a)
        cp_wlp = pltpu.async_copy(wlp_hbm.at[pl.ds(lo, rs)], wlp_dst, sem_b)

        def pscan(w, np_):
            pv = span_buf[pl.ds(w * 16, 16)]
            rv = lax.shift_right_arithmetic(pv, EID_BITS)
            ind = in_range01(rv, lo, lo + rs)
            ind = ind & in_range01(w * 16 + iota, 0, n_span)
            return compress(ind, pv, pass_buf, np_)

        n_pass = lax.fori_loop(0, (n_span + 15) // 16, pscan, 0)
        n_chunks = (n_pass + MC - 1) // MC

        def stage_chunk(cbase, cn):
            for g in range(MC // 16):
                pv = pass_buf[pl.ds(cbase + g * 16, 16)]
                indv = in_range01(g * 16 + iota, 0, cn)
                eid = pv & (B - 1)
                rl = lax.shift_right_arithmetic(pv, EID_BITS) - lo
                eid_buf[pl.ds(g * 16, 16)] = eid
                rloc_buf[pl.ds(g * 16, 16)] = rl * indv

        cp_wcl.wait()

        def sweep_a(ch, _):
            cbase = ch * MC
            cn = jnp.minimum(MC, n_pass - cbase)
            stage_chunk(cbase, cn)
            pltpu.async_copy(cp_hbm.at[eid_buf], cp_chunk, sem_c).wait()

            def ev_grp(g_, _c):
                rlv = rloc_buf[pl.ds(g_ * 16, 16)]
                for l in range(16):
                    @pl.when(g_ * 16 + l < cn)
                    def _(l=l):
                        r = rlv[l]
                        crow = cp_chunk.at[g_ * 16 + l]
                        for g4 in range(4):
                            plsc.addupdate(
                                wcl_chunk.at[r, pl.ds(g4 * 16, 16)],
                                crow[pl.ds(g4 * 16, 16)] * LR)
                return 0

            lax.fori_loop(0, (cn + 15) // 16, ev_grp, 0)
            return 0

        lax.fori_loop(0, n_chunks, sweep_a, 0)

        cp_wlp.wait()

        def sweep_b(ch, _):
            cbase = ch * MC
            cn = jnp.minimum(MC, n_pass - cbase)
            stage_chunk(cbase, cn)
            pltpu.async_copy(cp_hbm.at[eid_buf], cp_chunk, sem_c).wait()

            def grp(g_, _c):
                rl = rloc_buf[pl.ds(g_ * 16, 16)]
                slots = g_ * 16 + iota

                def col(c, acc):
                    cv = jnp.broadcast_to(c, (16,)).astype(jnp.int32)
                    a = plsc.load_gather(wcl_chunk, [rl, cv])
                    b = plsc.load_gather(cp_chunk, [slots, cv])
                    return acc + a * b

                acc = lax.fori_loop(0, N_CONCEPTS, col,
                                    jnp.zeros((16,), jnp.float32))
                gate_buf[pl.ds(g_ * 16, 16)] = jnp.maximum(
                    jnp.sign(acc - THETA_A), 0.0) * LR
                return 0

            lax.fori_loop(0, (cn + 15) // 16, grp, 0)

            def ev_grp(g_, _c):
                rlv = rloc_buf[pl.ds(g_ * 16, 16)]
                gvv = gate_buf[pl.ds(g_ * 16, 16)]
                for l in range(16):
                    @pl.when((g_ * 16 + l < cn) & (gvv[l] != 0.0))
                    def _(l=l):
                        r = rlv[l]
                        prow = cp_chunk.at[g_ * 16 + l]
                        for g4 in range(4):
                            plsc.addupdate(
                                wlp_chunk.at[r, pl.ds(g4 * 16, 16)],
                                prow[pl.ds(D_PHON + g4 * 16, 16)] * gvv[l])
                return 0

            lax.fori_loop(0, (cn + 15) // 16, ev_grp, 0)
            return 0

        lax.fori_loop(0, n_chunks, sweep_b, 0)

        pltpu.sync_copy(wlp_dst, out_hbm.at[pl.ds(lo, rs)])

    def pass_body(p, _):
        do_pass(span_lo + p * RS, RS)
        return 0

    lax.fori_loop(0, PASSES, pass_body, 0)

    @pl.when(is_last)
    def _():
        do_pass(jnp.int32(NW * ROWS_PER_W), TAIL)


@functools.partial(
    pl.kernel,
    out_type=jax.ShapeDtypeStruct((N_LEMMAS, D_PHON), jnp.float32),
    mesh=plsc.VectorSubcoreMesh(core_axis_name="c", subcore_axis_name="s"),
    compiler_params=pltpu.CompilerParams(needs_layout_passes=False),
    scratch_types=[
        pltpu.VMEM((IDX_BLK,), jnp.int32),
        pltpu.VMEM((B + 16,), jnp.int32),
        pltpu.VMEM((B + 16,), jnp.int32),
        pltpu.VMEM((MC,), jnp.int32),
        pltpu.VMEM((MC,), jnp.int32),
        pltpu.VMEM((MC,), jnp.float32),
        pltpu.VMEM((32,), jnp.int32),
        pltpu.VMEM((MC, N_CONCEPTS + D_PHON), jnp.float32),
        pltpu.VMEM((RS, N_CONCEPTS), jnp.float32),
        pltpu.VMEM((RS, D_PHON), jnp.float32),
        pltpu.SemaphoreType.DMA,
        pltpu.SemaphoreType.DMA,
        pltpu.SemaphoreType.DMA,
    ],
)
def _lemma_sc(idx_hbm, wcl_hbm, wlp_hbm, cp_hbm, out_hbm, *rest):
    _body(idx_hbm, wcl_hbm, wlp_hbm, cp_hbm, out_hbm, *rest)


def kernel(W_C_to_L, W_L_to_P, idx, concept, phon):
    cphon = jnp.concatenate([concept, phon], axis=1)
    return _lemma_sc(idx.astype(jnp.int32), W_C_to_L, W_L_to_P, cphon)


# speedup vs baseline: 1.1032x; 1.1032x over previous
import functools

import jax
import jax.numpy as jnp
from jax import lax
from jax.experimental import pallas as pl
from jax.experimental.pallas import tpu as pltpu
from jax.experimental.pallas import tpu_sc as plsc

N_LEMMAS = 100000
N_CONCEPTS = 64
D_PHON = 64
B = 16384
LR = 0.05
THETA_A = 0.3

NC = 2
NS = 16
NW = NC * NS
PASSES = 10
RS = 312
ROWS_PER_W = PASSES * RS
TAIL = N_LEMMAS - NW * ROWS_PER_W
MC = 64
EID_BITS = 14
IDX_BLK = 2048


def _body(idx_hbm, wcl_hbm, wlp_hbm, cp_hbm, out_hbm,
          idx_stage, span_buf, pass_buf, eid_buf, rloc_buf, gate_buf,
          cp_chunk, wcl_chunk, wlp_chunk,
          sem_a, sem_b, sem_c):
    wid = lax.axis_index("s") * NC + lax.axis_index("c")
    span_lo = wid * ROWS_PER_W
    is_last = wid == NW - 1
    span_hi = span_lo + ROWS_PER_W + (wid // (NW - 1)) * TAIL
    iota = lax.iota(jnp.int32, 16)

    def in_range01(x, lo, hi):
        d1 = lax.shift_right_arithmetic(x - lo, 31)
        d2 = lax.shift_right_arithmetic(hi - 1 - x, 31)
        return (1 + d1) & (1 + d2)

    def compress(ind, payload, dst, ns):
        m = ind > 0
        cnt = plsc.all_reduce_population_count(m)[0]

        @pl.when(cnt > 0)
        def _():
            plsc.store_compressed(dst.at[pl.ds(ns, 16)], payload, mask=m)

        return ns + cnt

    def blk_body(b, n_span):
        pltpu.sync_copy(idx_hbm.at[pl.ds(b * IDX_BLK, IDX_BLK)], idx_stage)

        def vec_body(v, ns):
            ivec = idx_stage[pl.ds(v * 16, 16)]
            ind = in_range01(ivec, span_lo, span_hi)
            packed = (ivec << EID_BITS) | (b * IDX_BLK + v * 16 + iota)
            return compress(ind, packed, span_buf, ns)

        return lax.fori_loop(0, IDX_BLK // 16, vec_body, n_span)

    n_span = lax.fori_loop(0, B // IDX_BLK, blk_body, 0)

    def do_pass(lo, rs):
        wcl_dst = wcl_chunk if rs == RS else wcl_chunk.at[pl.ds(0, rs)]
        wlp_dst = wlp_chunk if rs == RS else wlp_chunk.at[pl.ds(0, rs)]
        cp_wcl = pltpu.async_copy(wcl_hbm.at[pl.ds(lo, rs)], wcl_dst, sem_a)
        cp_wlp = pltpu.async_copy(wlp_hbm.at[pl.ds(lo, rs)], wlp_dst, sem_b)

        def pscan(w, np_):
            pv = span_buf[pl.ds(w * 16, 16)]
            rv = lax.shift_right_arithmetic(pv, EID_BITS)
            ind = in_range01(rv, lo, lo + rs)
            ind = ind & in_range01(w * 16 + iota, 0, n_span)
            return compress(ind, pv, pass_buf, np_)

        n_pass = lax.fori_loop(0, (n_span + 15) // 16, pscan, 0)
        n_chunks = (n_pass + MC - 1) // MC

        def stage_chunk(cbase, cn):
            for g in range(MC // 16):
                pv = pass_buf[pl.ds(cbase + g * 16, 16)]
                indv = in_range01(g * 16 + iota, 0, cn)
                eid = pv & (B - 1)
                rl = lax.shift_right_arithmetic(pv, EID_BITS) - lo
                eid_buf[pl.ds(g * 16, 16)] = eid
                rloc_buf[pl.ds(g * 16, 16)] = rl * indv

        cp_wcl.wait()

        def sweep_a(ch, _):
            cbase = ch * MC
            cn = jnp.minimum(MC, n_pass - cbase)
            stage_chunk(cbase, cn)
            pltpu.async_copy(cp_hbm.at[eid_buf], cp_chunk, sem_c).wait()

            def add_one(g_, l, rlv):
                crow = cp_chunk.at[g_ * 16 + l]
                for g4 in range(4):
                    plsc.addupdate(
                        wcl_chunk.at[rlv[l], pl.ds(g4 * 16, 16)],
                        crow[pl.ds(g4 * 16, 16)] * LR)

            n_full = cn // 16

            def ev_full(g_, _c):
                rlv = rloc_buf[pl.ds(g_ * 16, 16)]
                for l in range(16):
                    add_one(g_, l, rlv)
                return 0

            lax.fori_loop(0, n_full, ev_full, 0)

            @pl.when(n_full * 16 < cn)
            def _():
                rlv = rloc_buf[pl.ds(n_full * 16, 16)]
                for l in range(16):
                    @pl.when(n_full * 16 + l < cn)
                    def _(l=l):
                        add_one(n_full, l, rlv)
            return 0

        lax.fori_loop(0, n_chunks, sweep_a, 0)

        cp_wlp.wait()

        def sweep_b(ch, _):
            cbase = ch * MC
            cn = jnp.minimum(MC, n_pass - cbase)
            stage_chunk(cbase, cn)

            @pl.when(n_chunks > 1)
            def _():
                pltpu.async_copy(cp_hbm.at[eid_buf], cp_chunk, sem_c).wait()

            def grp(g_, _c):
                rl = rloc_buf[pl.ds(g_ * 16, 16)]
                slots = g_ * 16 + iota

                def col4(c4, accs):
                    outs = []
                    for k in range(4):
                        cv = jnp.broadcast_to(c4 * 4 + k, (16,)).astype(
                            jnp.int32)
                        av = plsc.load_gather(wcl_chunk, [rl, cv])
                        bv = plsc.load_gather(cp_chunk, [slots, cv])
                        outs.append(accs[k] + av * bv)
                    return tuple(outs)

                z = jnp.zeros((16,), jnp.float32)
                a0, a1, a2, a3 = lax.fori_loop(0, N_CONCEPTS // 4, col4,
                                               (z, z, z, z))
                acc = (a0 + a1) + (a2 + a3)
                gate_buf[pl.ds(g_ * 16, 16)] = jnp.maximum(
                    jnp.sign(acc - THETA_A), 0.0) * LR
                return 0

            lax.fori_loop(0, (cn + 15) // 16, grp, 0)

            def padd_one(g_, l, rlv, gvv):
                prow = cp_chunk.at[g_ * 16 + l]
                for g4 in range(4):
                    plsc.addupdate(
                        wlp_chunk.at[rlv[l], pl.ds(g4 * 16, 16)],
                        prow[pl.ds(D_PHON + g4 * 16, 16)] * gvv[l])

            n_full = cn // 16

            def pev_full(g_, _c):
                rlv = rloc_buf[pl.ds(g_ * 16, 16)]
                gvv = gate_buf[pl.ds(g_ * 16, 16)]
                for l in range(16):
                    padd_one(g_, l, rlv, gvv)
                return 0

            lax.fori_loop(0, n_full, pev_full, 0)

            @pl.when(n_full * 16 < cn)
            def _():
                rlv = rloc_buf[pl.ds(n_full * 16, 16)]
                gvv = gate_buf[pl.ds(n_full * 16, 16)]
                for l in range(16):
                    @pl.when(n_full * 16 + l < cn)
                    def _(l=l):
                        padd_one(n_full, l, rlv, gvv)
            return 0

        lax.fori_loop(0, n_chunks, sweep_b, 0)

        pltpu.sync_copy(wlp_dst, out_hbm.at[pl.ds(lo, rs)])

    def pass_body(p, _):
        do_pass(span_lo + p * RS, RS)
        return 0

    lax.fori_loop(0, PASSES, pass_body, 0)

    @pl.when(is_last)
    def _():
        do_pass(jnp.int32(NW * ROWS_PER_W), TAIL)


@functools.partial(
    pl.kernel,
    out_type=jax.ShapeDtypeStruct((N_LEMMAS, D_PHON), jnp.float32),
    mesh=plsc.VectorSubcoreMesh(core_axis_name="c", subcore_axis_name="s"),
    compiler_params=pltpu.CompilerParams(needs_layout_passes=False),
    scratch_types=[
        pltpu.VMEM((IDX_BLK,), jnp.int32),
        pltpu.VMEM((B + 16,), jnp.int32),
        pltpu.VMEM((B + 16,), jnp.int32),
        pltpu.VMEM((MC,), jnp.int32),
        pltpu.VMEM((MC,), jnp.int32),
        pltpu.VMEM((MC,), jnp.float32),
        pltpu.VMEM((MC, N_CONCEPTS + D_PHON), jnp.float32),
        pltpu.VMEM((RS, N_CONCEPTS), jnp.float32),
        pltpu.VMEM((RS, D_PHON), jnp.float32),
        pltpu.SemaphoreType.DMA,
        pltpu.SemaphoreType.DMA,
        pltpu.SemaphoreType.DMA,
    ],
)
def _lemma_sc(idx_hbm, wcl_hbm, wlp_hbm, cp_hbm, out_hbm, *rest):
    _body(idx_hbm, wcl_hbm, wlp_hbm, cp_hbm, out_hbm, *rest)


def kernel(W_C_to_L, W_L_to_P, idx, concept, phon):
    cphon = jnp.concatenate([concept, phon], axis=1)
    return _lemma_sc(idx.astype(jnp.int32), W_C_to_L, W_L_to_P, cphon)


# speedup vs baseline: 1.1141x; 1.0100x over previous
import functools

import jax
import jax.numpy as jnp
from jax import lax
from jax.experimental import pallas as pl
from jax.experimental.pallas import tpu as pltpu
from jax.experimental.pallas import tpu_sc as plsc

N_LEMMAS = 100000
N_CONCEPTS = 64
D_PHON = 64
B = 16384
LR = 0.05
THETA_A = 0.3

NC = 2
NS = 16
NW = NC * NS
PASSES = 10
RS = 312
ROWS_PER_W = PASSES * RS
TAIL = N_LEMMAS - NW * ROWS_PER_W
MC = 64
EID_BITS = 14
IDX_BLK = 2048


def _body(idx_hbm, wcl_hbm, wlp_hbm, cp_hbm, out_hbm,
          idx_stage, span_buf, pass_buf, eid_buf, rloc_buf, gate_buf,
          cp_chunk, wcl_chunk, wlp_chunk,
          sem_a, sem_b, sem_c):
    wid = lax.axis_index("s") * NC + lax.axis_index("c")
    span_lo = wid * ROWS_PER_W
    is_last = wid == NW - 1
    span_hi = span_lo + ROWS_PER_W + (wid // (NW - 1)) * TAIL
    iota = lax.iota(jnp.int32, 16)

    def in_range01(x, lo, hi):
        d1 = lax.shift_right_arithmetic(x - lo, 31)
        d2 = lax.shift_right_arithmetic(hi - 1 - x, 31)
        return (1 + d1) & (1 + d2)

    def compress(m, payload, dst, ns):
        cnt = plsc.all_reduce_population_count(m)[0]

        @pl.when(cnt > 0)
        def _():
            plsc.store_compressed(dst.at[pl.ds(ns, 16)], payload, mask=m)

        return ns + cnt

    def blk_body(b, n_span):
        pltpu.sync_copy(idx_hbm.at[pl.ds(b * IDX_BLK, IDX_BLK)], idx_stage)

        def vec_body(v, ns):
            ivec = idx_stage[pl.ds(v * 16, 16)]
            m = (ivec >= span_lo) & (ivec < span_hi)
            packed = (ivec << EID_BITS) | (b * IDX_BLK + v * 16 + iota)
            return compress(m, packed, span_buf, ns)

        return lax.fori_loop(0, IDX_BLK // 16, vec_body, n_span)

    n_span = lax.fori_loop(0, B // IDX_BLK, blk_body, 0)

    def do_pass(lo, rs):
        wcl_dst = wcl_chunk if rs == RS else wcl_chunk.at[pl.ds(0, rs)]
        wlp_dst = wlp_chunk if rs == RS else wlp_chunk.at[pl.ds(0, rs)]
        cp_wcl = pltpu.async_copy(wcl_hbm.at[pl.ds(lo, rs)], wcl_dst, sem_a)
        cp_wlp = pltpu.async_copy(wlp_hbm.at[pl.ds(lo, rs)], wlp_dst, sem_b)

        def pscan(w, np_):
            pv = span_buf[pl.ds(w * 16, 16)]
            rv = lax.shift_right_arithmetic(pv, EID_BITS)
            m = (rv >= lo) & (rv < lo + rs) & (w * 16 + iota < n_span)
            return compress(m, pv, pass_buf, np_)

        n_pass = lax.fori_loop(0, (n_span + 15) // 16, pscan, 0)
        n_chunks = (n_pass + MC - 1) // MC

        def stage_chunk(cbase, cn):
            for g in range(MC // 16):
                pv = pass_buf[pl.ds(cbase + g * 16, 16)]
                indv = in_range01(g * 16 + iota, 0, cn)
                eid = pv & (B - 1)
                rl = lax.shift_right_arithmetic(pv, EID_BITS) - lo
                eid_buf[pl.ds(g * 16, 16)] = eid
                rloc_buf[pl.ds(g * 16, 16)] = rl * indv

        cp_wcl.wait()

        def sweep_a(ch, _):
            cbase = ch * MC
            cn = jnp.minimum(MC, n_pass - cbase)
            stage_chunk(cbase, cn)
            pltpu.async_copy(cp_hbm.at[eid_buf], cp_chunk, sem_c).wait()

            def add_one(g_, l, rlv):
                crow = cp_chunk.at[g_ * 16 + l]
                for g4 in range(4):
                    plsc.addupdate(
                        wcl_chunk.at[rlv[l], pl.ds(g4 * 16, 16)],
                        crow[pl.ds(g4 * 16, 16)] * LR)

            n_full = cn // 16

            def ev_full(g_, _c):
                rlv = rloc_buf[pl.ds(g_ * 16, 16)]
                for l in range(16):
                    add_one(g_, l, rlv)
                return 0

            lax.fori_loop(0, n_full, ev_full, 0)

            @pl.when(n_full * 16 < cn)
            def _():
                rlv = rloc_buf[pl.ds(n_full * 16, 16)]
                for l in range(16):
                    @pl.when(n_full * 16 + l < cn)
                    def _(l=l):
                        add_one(n_full, l, rlv)
            return 0

        lax.fori_loop(0, n_chunks, sweep_a, 0)

        cp_wlp.wait()

        def sweep_b(ch, _):
            cbase = ch * MC
            cn = jnp.minimum(MC, n_pass - cbase)
            stage_chunk(cbase, cn)

            @pl.when(n_chunks > 1)
            def _():
                pltpu.async_copy(cp_hbm.at[eid_buf], cp_chunk, sem_c).wait()

            def grp(g_, _c):
                rl = rloc_buf[pl.ds(g_ * 16, 16)]
                slots = g_ * 16 + iota

                def col4(c4, accs):
                    outs = []
                    for k in range(4):
                        cv = jnp.broadcast_to(c4 * 4 + k, (16,)).astype(
                            jnp.int32)
                        av = plsc.load_gather(wcl_chunk, [rl, cv])
                        bv = plsc.load_gather(cp_chunk, [slots, cv])
                        outs.append(accs[k] + av * bv)
                    return tuple(outs)

                z = jnp.zeros((16,), jnp.float32)
                a0, a1, a2, a3 = lax.fori_loop(0, N_CONCEPTS // 4, col4,
                                               (z, z, z, z))
                acc = (a0 + a1) + (a2 + a3)
                gate_buf[pl.ds(g_ * 16, 16)] = jnp.maximum(
                    jnp.sign(acc - THETA_A), 0.0) * LR
                return 0

            lax.fori_loop(0, (cn + 15) // 16, grp, 0)

            def padd_one(g_, l, rlv, gvv):
                prow = cp_chunk.at[g_ * 16 + l]
                for g4 in range(4):
                    plsc.addupdate(
                        wlp_chunk.at[rlv[l], pl.ds(g4 * 16, 16)],
                        prow[pl.ds(D_PHON + g4 * 16, 16)] * gvv[l])

            n_full = cn // 16

            def pev_full(g_, _c):
                rlv = rloc_buf[pl.ds(g_ * 16, 16)]
                gvv = gate_buf[pl.ds(g_ * 16, 16)]
                for l in range(16):
                    padd_one(g_, l, rlv, gvv)
                return 0

            lax.fori_loop(0, n_full, pev_full, 0)

            @pl.when(n_full * 16 < cn)
            def _():
                rlv = rloc_buf[pl.ds(n_full * 16, 16)]
                gvv = gate_buf[pl.ds(n_full * 16, 16)]
                for l in range(16):
                    @pl.when(n_full * 16 + l < cn)
                    def _(l=l):
                        padd_one(n_full, l, rlv, gvv)
            return 0

        lax.fori_loop(0, n_chunks, sweep_b, 0)

        pltpu.sync_copy(wlp_dst, out_hbm.at[pl.ds(lo, rs)])

    def pass_body(p, _):
        do_pass(span_lo + p * RS, RS)
        return 0

    lax.fori_loop(0, PASSES, pass_body, 0)

    @pl.when(is_last)
    def _():
        do_pass(jnp.int32(NW * ROWS_PER_W), TAIL)


@functools.partial(
    pl.kernel,
    out_type=jax.ShapeDtypeStruct((N_LEMMAS, D_PHON), jnp.float32),
    mesh=plsc.VectorSubcoreMesh(core_axis_name="c", subcore_axis_name="s"),
    compiler_params=pltpu.CompilerParams(needs_layout_passes=False,
                                         skip_device_barrier=True),
    scratch_types=[
        pltpu.VMEM((IDX_BLK,), jnp.int32),
        pltpu.VMEM((B + 16,), jnp.int32),
        pltpu.VMEM((B + 16,), jnp.int32),
        pltpu.VMEM((MC,), jnp.int32),
        pltpu.VMEM((MC,), jnp.int32),
        pltpu.VMEM((MC,), jnp.float32),
        pltpu.VMEM((MC, N_CONCEPTS + D_PHON), jnp.float32),
        pltpu.VMEM((RS, N_CONCEPTS), jnp.float32),
        pltpu.VMEM((RS, D_PHON), jnp.float32),
        pltpu.SemaphoreType.DMA,
        pltpu.SemaphoreType.DMA,
        pltpu.SemaphoreType.DMA,
    ],
)
def _lemma_sc(idx_hbm, wcl_hbm, wlp_hbm, cp_hbm, out_hbm, *rest):
    _body(idx_hbm, wcl_hbm, wlp_hbm, cp_hbm, out_hbm, *rest)


def kernel(W_C_to_L, W_L_to_P, idx, concept, phon):
    cphon = jnp.concatenate([concept, phon], axis=1)
    return _lemma_sc(idx.astype(jnp.int32), W_C_to_L, W_L_to_P, cphon)
